# Initial kernel scaffold; baseline (speedup 1.0000x reference)
import jax
import jax.numpy as jnp
from jax.experimental import pallas as pl


def kernel(input_ids, wte):
    raise NotImplementedError("write your pallas kernel here")


# speedup vs baseline: 1.5569x; 1.5569x over previous
import functools

import jax
import jax.numpy as jnp
from jax import lax
from jax.experimental import pallas as pl
from jax.experimental.pallas import tpu as pltpu
from jax.experimental.pallas import tpu_sc as plsc

_D = 1024
_NC = 2
_NS = 16
_NW = _NC * _NS
_CH = 32


@functools.lru_cache(maxsize=None)
def _make_lookup(B, V, D):
    assert D == _D
    assert B % (_NW * _CH) == 0, B
    bpw = B // _NW
    nch = bpw // _CH

    mesh = plsc.VectorSubcoreMesh(core_axis_name="c", subcore_axis_name="s")

    @functools.partial(
        pl.kernel,
        mesh=mesh,
        out_type=jax.ShapeDtypeStruct((B, D), jnp.float32),
        scratch_types=[
            pltpu.VMEM((bpw,), jnp.int32),
            pltpu.VMEM((_CH, D), jnp.float32),
            pltpu.VMEM((_CH, D), jnp.float32),
            pltpu.SemaphoreType.DMA,
            pltpu.SemaphoreType.DMA,
        ],
    )
    def lookup(ids_hbm, table_hbm, out_hbm, idx_v, rows0, rows1, gsem, ssem):
        wid = lax.axis_index("s") * _NC + lax.axis_index("c")
        base = wid * bpw
        pltpu.sync_copy(ids_hbm.at[pl.ds(base, bpw)], idx_v)

        bufs = (rows0, rows1)
        gathers = [None, None]
        scatters = [None, None]
        for c in range(nch):
            s = c % 2
            if scatters[s] is not None:
                scatters[s].wait()
            gathers[s] = pltpu.async_copy(
                table_hbm.at[idx_v.at[pl.ds(c * _CH, _CH)]], bufs[s], gsem)
            if c >= 1:
                p = (c - 1) % 2
                gathers[p].wait()
                scatters[p] = pltpu.async_copy(
                    bufs[p], out_hbm.at[pl.ds(base + (c - 1) * _CH, _CH)],
                    ssem)
        last = (nch - 1) % 2
        gathers[last].wait()
        scatters[last] = pltpu.async_copy(
            bufs[last], out_hbm.at[pl.ds(base + (nch - 1) * _CH, _CH)], ssem)
        if nch >= 2:
            scatters[(nch - 2) % 2].wait()
        scatters[last].wait()

    return lookup


def kernel(input_ids, wte):
    in_shape = input_ids.shape
    ids = input_ids.reshape(-1).astype(jnp.int32)
    out = _make_lookup(ids.shape[0], wte.shape[0], wte.shape[1])(ids, wte)
    return out.reshape((-1, in_shape[-1], wte.shape[1]))


# speedup vs baseline: 1.5610x; 1.0027x over previous
import functools

import jax
import jax.numpy as jnp
from jax import lax
from jax.experimental import pallas as pl
from jax.experimental.pallas import tpu as pltpu
from jax.experimental.pallas import tpu_sc as plsc

_D = 1024
_NC = 2
_NS = 16
_NW = _NC * _NS
_CH = 32
_NBUF = 3


@functools.lru_cache(maxsize=None)
def _make_lookup(B, V, D):
    assert D == _D
    assert B % (_NW * _CH) == 0, B
    bpw = B // _NW
    nch = bpw // _CH

    mesh = plsc.VectorSubcoreMesh(core_axis_name="c", subcore_axis_name="s")

    @functools.partial(
        pl.kernel,
        mesh=mesh,
        out_type=jax.ShapeDtypeStruct((B, D), jnp.float32),
        scratch_types=[
            pltpu.VMEM((bpw,), jnp.int32),
        ] + [pltpu.VMEM((_CH, D), jnp.float32) for _ in range(_NBUF)] + [
            pltpu.SemaphoreType.DMA,
            pltpu.SemaphoreType.DMA,
        ],
    )
    def lookup(ids_hbm, table_hbm, out_hbm, idx_v, *rest):
        bufs = rest[:_NBUF]
        gsem, ssem = rest[_NBUF:]
        wid = lax.axis_index("s") * _NC + lax.axis_index("c")
        base = wid * bpw
        pltpu.sync_copy(ids_hbm.at[pl.ds(base, bpw)], idx_v)

        gathers = [None] * _NBUF
        scatters = [None] * _NBUF
        for c in range(nch):
            s = c % _NBUF
            if scatters[s] is not None:
                scatters[s].wait()
            gathers[s] = pltpu.async_copy(
                table_hbm.at[idx_v.at[pl.ds(c * _CH, _CH)]], bufs[s], gsem)
            if c >= 1:
                p = (c - 1) % _NBUF
                gathers[p].wait()
                scatters[p] = pltpu.async_copy(
                    bufs[p], out_hbm.at[pl.ds(base + (c - 1) * _CH, _CH)],
                    ssem)
        last = (nch - 1) % _NBUF
        gathers[last].wait()
        scatters[last] = pltpu.async_copy(
            bufs[last], out_hbm.at[pl.ds(base + (nch - 1) * _CH, _CH)], ssem)
        for c in range(max(0, nch - _NBUF), nch):
            scatters[c % _NBUF].wait()

    return lookup


def kernel(input_ids, wte):
    in_shape = input_ids.shape
    ids = input_ids.reshape(-1).astype(jnp.int32)
    out = _make_lookup(ids.shape[0], wte.shape[0], wte.shape[1])(ids, wte)
    return out.reshape((-1, in_shape[-1], wte.shape[1]))
